# baseline (speedup 1.0000x reference)
import jax
import jax.numpy as jnp
from jax.experimental import pallas as pl

_B = 256
_NL = 12
_KD = 768
_NT = 10
_NP = 8
_ED = 768
_NK = 100


def _layer_body(q_ref, keys_ref, p_ref, out_ref):
    q = q_ref[:, pl.program_id(0), :]
    keys = keys_ref[...]
    knorm = jax.lax.dot_general(
        jnp.ones((1, _KD), jnp.float32), keys * keys,
        (((1,), (1,)), ((), ())), preferred_element_type=jnp.float32)
    cross = jax.lax.dot_general(
        q, keys, (((1,), (1,)), ((), ())),
        preferred_element_type=jnp.float32)
    scores = knorm - 2.0 * cross
    colidx = jax.lax.broadcasted_iota(jnp.int32, (_B, _NK), 1)
    mval = jnp.min(scores, axis=1, keepdims=True)
    idx = jnp.min(jnp.where(scores == mval, colidx, _NK), axis=1, keepdims=True)
    task = idx // _NK
    onehot = (task == jax.lax.broadcasted_iota(jnp.int32, (_B, _NT), 1)
              ).astype(jnp.float32)
    for p in range(_NP):
        out_ref[0, :, p, :] = jax.lax.dot_general(
            onehot, p_ref[0, :, p, :], (((1,), (0,)), ((), ())),
            preferred_element_type=jnp.float32)


def kernel(x_query, vis_mark, P, task_keys):
    del vis_mark
    out = pl.pallas_call(
        _layer_body,
        grid=(_NL,),
        in_specs=[
            pl.BlockSpec((_B, _NL, _KD), lambda l: (0, 0, 0)),
            pl.BlockSpec((_NK, _KD), lambda l: (0, 0)),
            pl.BlockSpec((1, _NT, _NP, _ED), lambda l: (l, 0, 0, 0)),
        ],
        out_specs=pl.BlockSpec((1, _B, _NP, _ED), lambda l: (l, 0, 0, 0)),
        out_shape=jax.ShapeDtypeStruct((_NL, _B, _NP, _ED), jnp.float32),
    )(x_query, task_keys, P)
    return (out, jnp.float32(0.0))


# speedup vs baseline: 1.9475x; 1.9475x over previous
import jax
import jax.numpy as jnp
from jax.experimental import pallas as pl
from jax.experimental.pallas import tpu as pltpu

_B = 256
_NL = 12
_KD = 768
_NT = 10
_NP = 8
_ED = 768
_NK = 100
_BP = _B * _NP
_TP = _NT * _NP


def _layer_body(q_ref, keys_ref, p_ref, out_ref, e8_ref, cdiv_ref, mask_ref):
    @pl.when(pl.program_id(0) == 0)
    def _init():
        r = jax.lax.broadcasted_iota(jnp.int32, (_BP, _B), 0)
        b = jax.lax.broadcasted_iota(jnp.int32, (_BP, _B), 1)
        e8_ref[...] = ((r // _NP) == b).astype(jnp.float32)
        rp = jax.lax.broadcasted_iota(jnp.int32, (_BP, _TP), 0)
        cp = jax.lax.broadcasted_iota(jnp.int32, (_BP, _TP), 1)
        cdiv_ref[...] = cp // _NP
        mask_ref[...] = ((rp % _NP) == (cp % _NP)).astype(jnp.float32)

    q = q_ref[0]
    keys = keys_ref[...]
    knorm = jax.lax.dot_general(
        jnp.ones((1, _KD), jnp.float32), keys * keys,
        (((1,), (1,)), ((), ())), preferred_element_type=jnp.float32)
    cross = jax.lax.dot_general(
        q, keys, (((1,), (1,)), ((), ())),
        preferred_element_type=jnp.float32)
    scores = knorm - 2.0 * cross
    colidx = jax.lax.broadcasted_iota(jnp.int32, (_B, _NK), 1)
    mval = jnp.min(scores, axis=1, keepdims=True)
    idx = jnp.min(jnp.where(scores == mval, colidx, _NK), axis=1, keepdims=True)
    taskf = (idx // _NK).astype(jnp.float32)
    t2048 = jax.lax.dot_general(
        e8_ref[...], taskf, (((1,), (0,)), ((), ())),
        preferred_element_type=jnp.float32)
    onehot2 = jnp.where(t2048.astype(jnp.int32) == cdiv_ref[...],
                        mask_ref[...], 0.0)
    res = jax.lax.dot_general(
        onehot2, p_ref[0], (((1,), (0,)), ((), ())),
        preferred_element_type=jnp.float32)
    out_ref[0] = res.reshape(_B, _NP, _ED)


def kernel(x_query, vis_mark, P, task_keys):
    del vis_mark
    p2 = P.reshape(_NL, _TP, _ED)
    xq = jnp.transpose(x_query, (1, 0, 2))
    out = pl.pallas_call(
        _layer_body,
        grid=(_NL,),
        in_specs=[
            pl.BlockSpec((1, _B, _KD), lambda l: (l, 0, 0)),
            pl.BlockSpec((_NK, _KD), lambda l: (0, 0)),
            pl.BlockSpec((1, _TP, _ED), lambda l: (l, 0, 0)),
        ],
        out_specs=pl.BlockSpec((1, _B, _NP, _ED), lambda l: (l, 0, 0, 0)),
        out_shape=jax.ShapeDtypeStruct((_NL, _B, _NP, _ED), jnp.float32),
        scratch_shapes=[
            pltpu.VMEM((_BP, _B), jnp.float32),
            pltpu.VMEM((_BP, _TP), jnp.int32),
            pltpu.VMEM((_BP, _TP), jnp.float32),
        ],
    )(xq, task_keys, p2)
    return (out, jnp.float32(0.0))
